# baseline (speedup 1.0000x reference)
import jax
import jax.numpy as jnp
from jax import lax
from jax.experimental import pallas as pl
from jax.experimental.pallas import tpu as pltpu
from jax.experimental.pallas import tpu_sc as plsc

VOCAB_SIZES = [99991, 100003, 100019, 100043, 100049, 100057, 100069, 100103]
H = len(VOCAB_SIZES)
D = 64
B, S = 4, 2048
N = B * S * H

_off = []
_acc = 0
for _v in VOCAB_SIZES:
    _off.append(_acc)
    _acc += _v
OFF16 = tuple(_off * 2)

NC, NS, L = 2, 16, 16
NW = NC * NS
SW = S * B // NW
PER_W = SW * H
CHUNK = 128
SC_CHUNK = CHUNK // H
NCHUNK = PER_W // CHUNK

NBUF = 4
DEPTH = 2


def _body(ids_hbm, table_hbm, off_hbm, out_hbm, stag_v, bufs_v, off_v, *sems):
    gsems = sems[:NBUF]
    wsems = sems[NBUF:]
    wid = lax.axis_index("s") * NC + lax.axis_index("c")
    base = wid * PER_W

    pltpu.sync_copy(ids_hbm.at[pl.ds(base, PER_W)], stag_v)

    pltpu.sync_copy(off_hbm, off_v)
    off = off_v[...]

    def add_step(k, _):
        sl = pl.ds(k * L, L)
        stag_v[sl] = stag_v[sl] + off
        return 0

    lax.fori_loop(0, PER_W // L, add_step, 0)

    g = [None] * NCHUNK
    w = [None] * NCHUNK

    def retire(j):
        g[j].wait()
        w[j] = pltpu.async_copy(
            bufs_v.at[j % NBUF],
            out_hbm.at[pl.ds(base + j * CHUNK, CHUNK)],
            wsems[j % NBUF],
        )

    for j in range(NCHUNK):
        bi = j % NBUF
        if j >= NBUF:
            w[j - NBUF].wait()
        g[j] = pltpu.async_copy(
            table_hbm.at[stag_v.at[pl.ds(j * CHUNK, CHUNK)]],
            bufs_v.at[bi],
            gsems[bi],
        )
        if j >= DEPTH:
            retire(j - DEPTH)
    for j in range(NCHUNK - DEPTH, NCHUNK):
        retire(j)
    for j in range(NCHUNK - NBUF, NCHUNK):
        w[j].wait()


@jax.jit
def kernel(input_ids, table):
    off16 = jnp.asarray(OFF16, dtype=jnp.int32)
    mesh = plsc.VectorSubcoreMesh(core_axis_name="c", subcore_axis_name="s")
    ids1d = input_ids.reshape(N)
    out = pl.kernel(
        _body,
        mesh=mesh,
        out_type=jax.ShapeDtypeStruct((N, D), jnp.float32),
        compiler_params=pltpu.CompilerParams(use_tc_tiling_on_sc=False),
        scratch_types=[
            pltpu.VMEM((PER_W,), jnp.int32),
            pltpu.VMEM((NBUF, CHUNK, D), jnp.float32),
            pltpu.VMEM((L,), jnp.int32),
        ]
        + [pltpu.SemaphoreType.DMA] * (2 * NBUF),
    )(ids1d, table, off16)
    return out.reshape(B, S, H, D)
